# baseline (speedup 1.0000x reference)
import jax
import jax.numpy as jnp
from jax import lax
from jax.experimental import pallas as pl
from jax.experimental.pallas import tpu as pltpu
from jax.experimental.pallas import tpu_sc as plsc

N_NODES = 10000
D_FEAT = 128
N_LAB = 16
LABW = 24
N_PAD = 10112
STRIPE = N_PAD // 16
CHUNK = 48
CPT = 420
NSUB = 16
E_PAD = NSUB * CPT * CHUNK


def _sc_body(x_hbm, idx2_hbm, lab_hbm, zx_hbm, zl_hbm,
             sx_out, sl_out,
             idx_v, rows_v, lraw_v, sbuf_v, acc_x, acc_l,
             isem0, isem1, isem2, isem3, gsem0, gsem1, gsem2, gsem3,
             lcsem0, lcsem1, lcsem2, lcsem3, sxsem0, sxsem1, sxsem2, sxsem3,
             slsem0, slsem1, slsem2, slsem3):
    c = lax.axis_index("c")
    t = lax.axis_index("s")
    stripe = t * STRIPE
    isem = (isem0, isem1, isem2, isem3)
    gsem = (gsem0, gsem1, gsem2, gsem3)
    lcsem = (lcsem0, lcsem1, lcsem2, lcsem3)
    sxsem = (sxsem0, sxsem1, sxsem2, sxsem3)
    slsem = (slsem0, slsem1, slsem2, slsem3)

    pltpu.sync_copy(zx_hbm, acc_x.at[pl.ds(stripe, STRIPE)])
    pltpu.sync_copy(zl_hbm, acc_l.at[pl.ds(stripe, STRIPE)])
    plsc.subcore_barrier()

    tail = jnp.where(lax.broadcasted_iota(jnp.int32, (16,), 0) == 8,
                     jnp.float32(1.0), jnp.float32(0.0))
    for _b in range(4):
        for _e in range(CHUNK):
            sbuf_v[_b, _e, 8:24] = tail

    LROWS = CHUNK // 8

    def load_idx(i, b):
        pltpu.async_copy(idx2_hbm.at[c * NSUB * CPT + t * CPT + i],
                         idx_v.at[b], isem[b])

    def start_fetch(i, b):
        pltpu.make_async_copy(idx2_hbm.at[c * NSUB * CPT + t * CPT + i],
                              idx_v.at[b], isem[b]).wait()
        pltpu.async_copy(x_hbm.at[idx_v.at[b, 0]], rows_v.at[b], gsem[b])
        lbase = (t * CPT + i) * LROWS
        pltpu.async_copy(lab_hbm.at[pl.ds(lbase, LROWS)], lraw_v.at[b],
                         lcsem[b])

    def wait_scatters(b):
        pltpu.make_async_copy(rows_v.at[b], acc_x.at[idx_v.at[b, 1]],
                              sxsem[b]).wait()
        pltpu.make_async_copy(sbuf_v.at[b], acc_l.at[idx_v.at[b, 1]],
                              slsem[b]).wait()

    def finish(i, b):
        lbase = (t * CPT + i) * LROWS
        pltpu.make_async_copy(x_hbm.at[idx_v.at[b, 0]], rows_v.at[b],
                              gsem[b]).wait()
        pltpu.make_async_copy(lab_hbm.at[pl.ds(lbase, LROWS)], lraw_v.at[b],
                              lcsem[b]).wait()
        pltpu.async_copy(rows_v.at[b], acc_x.at[idx_v.at[b, 1]], sxsem[b],
                         add=True)
        for _e in range(CHUNK):
            sbuf_v[b, _e, 0:16] = lraw_v[b, _e // 8,
                                         (_e % 8) * 16:(_e % 8) * 16 + 16]
        pltpu.async_copy(sbuf_v.at[b], acc_l.at[idx_v.at[b, 1]], slsem[b],
                         add=True)

    load_idx(0, 0)
    load_idx(1, 1)
    start_fetch(0, 0)
    load_idx(2, 2)
    start_fetch(1, 1)
    finish(0, 0)
    load_idx(3, 3)
    start_fetch(2, 2)
    finish(1, 1)

    def quad_body(j, carry):
        i0 = 4 * j
        wait_scatters(0)
        load_idx(i0, 0)
        start_fetch(i0 - 1, 3)
        finish(i0 - 2, 2)
        wait_scatters(1)
        load_idx(i0 + 1, 1)
        start_fetch(i0, 0)
        finish(i0 - 1, 3)
        wait_scatters(2)
        load_idx(i0 + 2, 2)
        start_fetch(i0 + 1, 1)
        finish(i0, 0)
        wait_scatters(3)
        load_idx(i0 + 3, 3)
        start_fetch(i0 + 2, 2)
        finish(i0 + 1, 1)
        return carry

    lax.fori_loop(1, CPT // 4, quad_body, 0)
    start_fetch(CPT - 1, 3)
    finish(CPT - 2, 2)
    finish(CPT - 1, 3)
    wait_scatters(0)
    wait_scatters(1)
    wait_scatters(2)
    wait_scatters(3)
    plsc.subcore_barrier()

    pltpu.sync_copy(acc_x.at[pl.ds(stripe, STRIPE)],
                    sx_out.at[c, pl.ds(stripe, STRIPE)])
    pltpu.sync_copy(acc_l.at[pl.ds(stripe, STRIPE)],
                    sl_out.at[c, pl.ds(stripe, STRIPE)])


def _tc_body(sxr, slr, sxc, slc, w, tw, bb, o):
    icr = 1.0 / jnp.maximum(slr[:, 16:17], 1.0)
    icc = 1.0 / jnp.maximum(slc[:, 16:17], 1.0)
    w0 = w[0:128]
    w1 = w[128:256]
    w2 = w[256:384]
    w3 = w[384:512]
    wl1 = jnp.dot(tw[...], w1, preferred_element_type=jnp.float32)
    wl3 = jnp.dot(tw[...], w3, preferred_element_type=jnp.float32)
    acc = jnp.dot(sxr[...] * icr, w0, preferred_element_type=jnp.float32)
    acc += jnp.dot(slr[:, :N_LAB] * icr, wl1, preferred_element_type=jnp.float32)
    acc += jnp.dot(sxc[...] * icc, w2, preferred_element_type=jnp.float32)
    acc += jnp.dot(slc[:, :N_LAB] * icc, wl3, preferred_element_type=jnp.float32)
    o[...] = acc + bb[0:1, :]


def kernel(x, edge_index, edge_label, W, trans_W, b):
    e = edge_index.shape[1]
    pad = E_PAD - e
    row = edge_index[0].astype(jnp.int32)
    col = edge_index[1].astype(jnp.int32)
    rowp = jnp.concatenate([row, jnp.full((pad,), N_NODES, jnp.int32)])
    colp = jnp.concatenate([col, jnp.full((pad,), N_NODES, jnp.int32)])
    rowg = jnp.concatenate([row, jnp.zeros((pad,), jnp.int32)])
    colg = jnp.concatenate([col, jnp.zeros((pad,), jnp.int32)])

    idx2 = jnp.concatenate([
        jnp.stack([colg.reshape(NSUB * CPT, CHUNK),
                   rowp.reshape(NSUB * CPT, CHUNK)], axis=1),
        jnp.stack([rowg.reshape(NSUB * CPT, CHUNK),
                   colp.reshape(NSUB * CPT, CHUNK)], axis=1),
    ], axis=0)
    lab = jnp.concatenate(
        [edge_label, jnp.zeros((pad, N_LAB), jnp.float32)]
    ).reshape(E_PAD // 8, 8 * N_LAB)
    zx = jnp.zeros((STRIPE, D_FEAT), jnp.float32)
    zl = jnp.zeros((STRIPE, LABW), jnp.float32)

    mesh = plsc.VectorSubcoreMesh(core_axis_name="c", subcore_axis_name="s")
    sc = pl.kernel(
        _sc_body,
        out_type=[
            jax.ShapeDtypeStruct((2, N_PAD, D_FEAT), jnp.float32),
            jax.ShapeDtypeStruct((2, N_PAD, LABW), jnp.float32),
        ],
        mesh=mesh,
        scratch_types=[
            pltpu.VMEM((4, 2, CHUNK), jnp.int32),
            pltpu.VMEM((4, CHUNK, D_FEAT), jnp.float32),
            pltpu.VMEM((4, CHUNK // 8, 8 * N_LAB), jnp.float32),
            pltpu.VMEM((4, CHUNK, LABW), jnp.float32),
            pltpu.VMEM_SHARED((N_PAD, D_FEAT), jnp.float32),
            pltpu.VMEM_SHARED((N_PAD, LABW), jnp.float32),
        ] + [pltpu.SemaphoreType.DMA] * 20,
        compiler_params=pltpu.CompilerParams(use_tc_tiling_on_sc=False),
    )
    sx, sl = sc(x, idx2, lab, zx, zl)

    blk = STRIPE
    nblk = N_PAD // blk
    out = pl.pallas_call(
        _tc_body,
        grid=(nblk,),
        in_specs=[
            pl.BlockSpec((blk, D_FEAT), lambda i: (i, 0)),
            pl.BlockSpec((blk, LABW), lambda i: (i, 0)),
            pl.BlockSpec((blk, D_FEAT), lambda i: (i, 0)),
            pl.BlockSpec((blk, LABW), lambda i: (i, 0)),
            pl.BlockSpec((512, D_FEAT), lambda i: (0, 0)),
            pl.BlockSpec((N_LAB, D_FEAT), lambda i: (0, 0)),
            pl.BlockSpec((1, D_FEAT), lambda i: (0, 0)),
        ],
        out_specs=pl.BlockSpec((blk, D_FEAT), lambda i: (i, 0)),
        out_shape=jax.ShapeDtypeStruct((N_PAD, D_FEAT), jnp.float32),
    )(sx[0], sl[0], sx[1], sl[1], W, trans_W, b.reshape(1, D_FEAT))
    return out[:N_NODES]


# speedup vs baseline: 1.4764x; 1.4764x over previous
import jax
import jax.numpy as jnp
from jax import lax
from jax.experimental import pallas as pl
from jax.experimental.pallas import tpu as pltpu
from jax.experimental.pallas import tpu_sc as plsc

N_NODES = 10000
D_FEAT = 128
N_LAB = 16
LABW = 32
N_PAD = 10112
STRIPE = N_PAD // 16
CHUNK = 40
CPT = 500
NSUB = 16
E_PAD = NSUB * CPT * CHUNK


def _sc_body(x_hbm, idx2_hbm, lab_hbm, zx_hbm, zl_hbm,
             sx_out, sl_out,
             idx_v, rows_v, lraw_v, sbuf_v, acc_x, acc_l,
             isem0, isem1, isem2, isem3, gsem0, gsem1, gsem2, gsem3,
             lcsem0, lcsem1, lcsem2, lcsem3, sxsem0, sxsem1, sxsem2, sxsem3,
             slsem0, slsem1, slsem2, slsem3):
    c = lax.axis_index("c")
    t = lax.axis_index("s")
    stripe = t * STRIPE
    isem = (isem0, isem1, isem2, isem3)
    gsem = (gsem0, gsem1, gsem2, gsem3)
    lcsem = (lcsem0, lcsem1, lcsem2, lcsem3)
    sxsem = (sxsem0, sxsem1, sxsem2, sxsem3)
    slsem = (slsem0, slsem1, slsem2, slsem3)

    pltpu.sync_copy(zx_hbm, acc_x.at[pl.ds(stripe, STRIPE)])
    pltpu.sync_copy(zl_hbm, acc_l.at[pl.ds(stripe, STRIPE)])
    plsc.subcore_barrier()

    tail = jnp.where(lax.broadcasted_iota(jnp.int32, (16,), 0) == 0,
                     jnp.float32(1.0), jnp.float32(0.0))
    for _b in range(4):
        for _e in range(CHUNK):
            sbuf_v[_b, _e, 16:32] = tail

    LROWS = CHUNK // 8

    def load_idx(i, b):
        pltpu.async_copy(idx2_hbm.at[c * NSUB * CPT + t * CPT + i],
                         idx_v.at[b], isem[b])

    def start_fetch(i, b):
        pltpu.make_async_copy(idx2_hbm.at[c * NSUB * CPT + t * CPT + i],
                              idx_v.at[b], isem[b]).wait()
        pltpu.async_copy(x_hbm.at[idx_v.at[b, 0]], rows_v.at[b], gsem[b])
        lbase = (t * CPT + i) * LROWS
        pltpu.async_copy(lab_hbm.at[pl.ds(lbase, LROWS)], lraw_v.at[b],
                         lcsem[b])

    def wait_scatters(b):
        pltpu.make_async_copy(rows_v.at[b], acc_x.at[idx_v.at[b, 1]],
                              sxsem[b]).wait()
        pltpu.make_async_copy(sbuf_v.at[b], acc_l.at[idx_v.at[b, 1]],
                              slsem[b]).wait()

    def finish(i, b):
        lbase = (t * CPT + i) * LROWS
        pltpu.make_async_copy(x_hbm.at[idx_v.at[b, 0]], rows_v.at[b],
                              gsem[b]).wait()
        pltpu.make_async_copy(lab_hbm.at[pl.ds(lbase, LROWS)], lraw_v.at[b],
                              lcsem[b]).wait()
        pltpu.async_copy(rows_v.at[b], acc_x.at[idx_v.at[b, 1]], sxsem[b],
                         add=True)
        for _e in range(CHUNK):
            sbuf_v[b, _e, 0:16] = lraw_v[b, _e // 8,
                                         (_e % 8) * 16:(_e % 8) * 16 + 16]
        pltpu.async_copy(sbuf_v.at[b], acc_l.at[idx_v.at[b, 1]], slsem[b],
                         add=True)

    load_idx(0, 0)
    load_idx(1, 1)
    start_fetch(0, 0)
    load_idx(2, 2)
    start_fetch(1, 1)
    finish(0, 0)
    load_idx(3, 3)
    start_fetch(2, 2)
    finish(1, 1)

    def quad_body(j, carry):
        i0 = 4 * j
        wait_scatters(0)
        load_idx(i0, 0)
        start_fetch(i0 - 1, 3)
        finish(i0 - 2, 2)
        wait_scatters(1)
        load_idx(i0 + 1, 1)
        start_fetch(i0, 0)
        finish(i0 - 1, 3)
        wait_scatters(2)
        load_idx(i0 + 2, 2)
        start_fetch(i0 + 1, 1)
        finish(i0, 0)
        wait_scatters(3)
        load_idx(i0 + 3, 3)
        start_fetch(i0 + 2, 2)
        finish(i0 + 1, 1)
        return carry

    lax.fori_loop(1, CPT // 4, quad_body, 0)
    start_fetch(CPT - 1, 3)
    finish(CPT - 2, 2)
    finish(CPT - 1, 3)
    wait_scatters(0)
    wait_scatters(1)
    wait_scatters(2)
    wait_scatters(3)
    plsc.subcore_barrier()

    pltpu.sync_copy(acc_x.at[pl.ds(stripe, STRIPE)],
                    sx_out.at[c, pl.ds(stripe, STRIPE)])
    pltpu.sync_copy(acc_l.at[pl.ds(stripe, STRIPE)],
                    sl_out.at[c, pl.ds(stripe, STRIPE)])


def _tc_body(sxr, slr, sxc, slc, w, tw, bb, o):
    icr = 1.0 / jnp.maximum(slr[:, 16:17], 1.0)
    icc = 1.0 / jnp.maximum(slc[:, 16:17], 1.0)
    w0 = w[0:128]
    w1 = w[128:256]
    w2 = w[256:384]
    w3 = w[384:512]
    wl1 = jnp.dot(tw[...], w1, preferred_element_type=jnp.float32)
    wl3 = jnp.dot(tw[...], w3, preferred_element_type=jnp.float32)
    acc = jnp.dot(sxr[...] * icr, w0, preferred_element_type=jnp.float32)
    acc += jnp.dot(slr[:, :N_LAB] * icr, wl1, preferred_element_type=jnp.float32)
    acc += jnp.dot(sxc[...] * icc, w2, preferred_element_type=jnp.float32)
    acc += jnp.dot(slc[:, :N_LAB] * icc, wl3, preferred_element_type=jnp.float32)
    o[...] = acc + bb[0:1, :]


def kernel(x, edge_index, edge_label, W, trans_W, b):
    e = edge_index.shape[1]
    pad = E_PAD - e
    row = edge_index[0].astype(jnp.int32)
    col = edge_index[1].astype(jnp.int32)
    rowp = jnp.concatenate([row, jnp.full((pad,), N_NODES, jnp.int32)])
    colp = jnp.concatenate([col, jnp.full((pad,), N_NODES, jnp.int32)])
    rowg = jnp.concatenate([row, jnp.zeros((pad,), jnp.int32)])
    colg = jnp.concatenate([col, jnp.zeros((pad,), jnp.int32)])

    idx2 = jnp.concatenate([
        jnp.stack([colg.reshape(NSUB * CPT, CHUNK),
                   rowp.reshape(NSUB * CPT, CHUNK)], axis=1),
        jnp.stack([rowg.reshape(NSUB * CPT, CHUNK),
                   colp.reshape(NSUB * CPT, CHUNK)], axis=1),
    ], axis=0)
    lab = edge_label.reshape(E_PAD // 8, 8 * N_LAB)
    zx = jnp.zeros((STRIPE, D_FEAT), jnp.float32)
    zl = jnp.zeros((STRIPE, LABW), jnp.float32)

    mesh = plsc.VectorSubcoreMesh(core_axis_name="c", subcore_axis_name="s")
    sc = pl.kernel(
        _sc_body,
        out_type=[
            jax.ShapeDtypeStruct((2, N_PAD, D_FEAT), jnp.float32),
            jax.ShapeDtypeStruct((2, N_PAD, LABW), jnp.float32),
        ],
        mesh=mesh,
        scratch_types=[
            pltpu.VMEM((4, 2, CHUNK), jnp.int32),
            pltpu.VMEM((4, CHUNK, D_FEAT), jnp.float32),
            pltpu.VMEM((4, CHUNK // 8, 8 * N_LAB), jnp.float32),
            pltpu.VMEM((4, CHUNK, LABW), jnp.float32),
            pltpu.VMEM_SHARED((N_PAD, D_FEAT), jnp.float32),
            pltpu.VMEM_SHARED((N_PAD, LABW), jnp.float32),
        ] + [pltpu.SemaphoreType.DMA] * 20,
        compiler_params=pltpu.CompilerParams(use_tc_tiling_on_sc=False),
    )
    sx, sl = sc(x, idx2, lab, zx, zl)

    blk = STRIPE
    nblk = N_PAD // blk
    out = pl.pallas_call(
        _tc_body,
        grid=(nblk,),
        in_specs=[
            pl.BlockSpec((blk, D_FEAT), lambda i: (i, 0)),
            pl.BlockSpec((blk, LABW), lambda i: (i, 0)),
            pl.BlockSpec((blk, D_FEAT), lambda i: (i, 0)),
            pl.BlockSpec((blk, LABW), lambda i: (i, 0)),
            pl.BlockSpec((512, D_FEAT), lambda i: (0, 0)),
            pl.BlockSpec((N_LAB, D_FEAT), lambda i: (0, 0)),
            pl.BlockSpec((1, D_FEAT), lambda i: (0, 0)),
        ],
        out_specs=pl.BlockSpec((blk, D_FEAT), lambda i: (i, 0)),
        out_shape=jax.ShapeDtypeStruct((N_PAD, D_FEAT), jnp.float32),
    )(sx[0], sl[0], sx[1], sl[1], W, trans_W, b.reshape(1, D_FEAT))
    return out[:N_NODES]
